# baseline (speedup 1.0000x reference)
import functools

import jax
import jax.numpy as jnp
import numpy as np
from jax import lax
from jax.experimental import pallas as pl
from jax.experimental.pallas import tpu as pltpu
from jax.experimental.pallas import tpu_sc as plsc

N = 10000
E = 160000
G = 64
C = 256

NW = 32
L = 16

KR = 4000
NCH_R = E // KR
CBUF = 4096
CAP = 164864
PADROW = 326
NROW = 328
NPAD = 10240
CCH = 768
BLK = 64

@functools.lru_cache(maxsize=None)
def _mesh():
    return plsc.VectorSubcoreMesh(core_axis_name="c", subcore_axis_name="s")


def _wid():
    return lax.axis_index("s") * 2 + lax.axis_index("c")


def _make_route(has_w):
    def body(*refs):
        if has_w:
            (src_h, dst_h, w_h, cnt_h, es_h, ed_h, ew_h,
             ssrc, sdst, sw, csrc, cdst, cw, cntw) = refs
        else:
            (src_h, dst_h, cnt_h, es_h, ed_h,
             ssrc, sdst, csrc, cdst, cntw) = refs
            sw = cw = ew_h = None
        wid = _wid()
        zi = jnp.zeros((L,), jnp.int32)
        def zb(i, _):
            csrc[pl.ds(i * L, L)] = zi
            cdst[pl.ds(i * L, L)] = zi
            return 0
        lax.fori_loop(0, CBUF // L, zb, 0)

        padrow = jnp.full((L,), PADROW, jnp.int32)
        lanes = lax.broadcasted_iota(jnp.int32, (L,), 0)

        def chunk(ch, written):
            base = ch * KR
            pltpu.sync_copy(src_h.at[pl.ds(base, KR)], ssrc)
            pltpu.sync_copy(dst_h.at[pl.ds(base, KR)], sdst)
            if has_w:
                pltpu.sync_copy(w_h.at[pl.ds(base, KR)], sw)

            def group(g, pos):
                s = ssrc[pl.ds(g * L, L)]
                d = sdst[pl.ds(g * L, L)]
                own = (d >> 5) & 31
                m = own == wid
                dl = ((d >> 10) << 5) | (d & 31)
                mi = m.astype(jnp.int32)
                rank = plsc.cumsum(mi) - 1
                tgt = jnp.where(m, pos + rank, CBUF + lanes)
                plsc.store_scatter(csrc, [tgt], s)
                plsc.store_scatter(cdst, [tgt], dl)
                if has_w:
                    wv = sw[pl.ds(g * L, L)]
                    plsc.store_scatter(cw, [tgt], wv)
                c = plsc.all_reduce_population_count(m)[0]
                return pos + c

            pos = lax.fori_loop(0, KR // L, group, 0)
            pad = (-pos) & 15
            pm = lanes < pad
            tgt = jnp.where(pm, pos + lanes, CBUF + lanes)
            plsc.store_scatter(csrc, [tgt], zi)
            plsc.store_scatter(cdst, [tgt], padrow)
            if has_w:
                plsc.store_scatter(cw, [tgt], jnp.zeros((L,), jnp.float32))
            pos = pos + pad
            lbase = pl.multiple_of(wid * CAP + written, 16)
            pltpu.sync_copy(csrc.at[pl.ds(0, CBUF)], es_h.at[pl.ds(lbase, CBUF)])
            pltpu.sync_copy(cdst.at[pl.ds(0, CBUF)], ed_h.at[pl.ds(lbase, CBUF)])
            if has_w:
                pltpu.sync_copy(cw.at[pl.ds(0, CBUF)], ew_h.at[pl.ds(lbase, CBUF)])
            return written + pos

        total = lax.fori_loop(0, NCH_R, chunk, 0)
        cntw[...] = jnp.full((L,), total, jnp.int32)
        pltpu.sync_copy(cntw, cnt_h.at[pl.ds(pl.multiple_of(wid * L, 16), L)])

    out_type = [
        jax.ShapeDtypeStruct((NW * L,), jnp.int32),
        jax.ShapeDtypeStruct((NW * CAP,), jnp.int32),
        jax.ShapeDtypeStruct((NW * CAP,), jnp.int32),
    ]
    scratch = [
        pltpu.VMEM((KR,), jnp.int32),
        pltpu.VMEM((KR,), jnp.int32),
    ]
    if has_w:
        out_type.append(jax.ShapeDtypeStruct((NW * CAP,), jnp.float32))
        scratch.append(pltpu.VMEM((KR,), jnp.float32))
    scratch += [
        pltpu.VMEM((CBUF + L,), jnp.int32),
        pltpu.VMEM((CBUF + L,), jnp.int32),
    ]
    if has_w:
        scratch.append(pltpu.VMEM((CBUF + L,), jnp.float32))
    scratch.append(pltpu.VMEM((L,), jnp.int32))
    return pl.kernel(body, out_type=tuple(out_type), mesh=_mesh(),
                     compiler_params=pltpu.CompilerParams(needs_layout_passes=False),
                     scratch_types=tuple(scratch))


def _make_conv(has_w):
    def body(*refs):
        if has_w:
            (h_h, asrc_h, adst_h, cnt_h, es_h, ed_h, ew_h, out_h,
             asrc_v, adst_v, denom, acc, esv, edv, ewv, coefv, rows,
             cntv, sem, sem2, sem3, sem4) = refs
        else:
            (h_h, asrc_h, adst_h, cnt_h, es_h, ed_h, out_h,
             asrc_v, adst_v, denom, acc, esv, edv, coefv, rows,
             cntv, sem, sem2, sem3, sem4) = refs
            ewv = ew_h = None
        wid = _wid()
        pltpu.sync_copy(cnt_h, cntv)
        pltpu.sync_copy(asrc_h, asrc_v)
        for jj in range(10):
            pltpu.sync_copy(adst_h.at[pl.ds((jj * 32 + wid) * 32, 32)],
                            adst_v.at[pl.ds(jj * 32, 32)])
        zf = jnp.zeros((L,), jnp.float32)

        def zacc(r, _):
            for k in range(C // L):
                acc[r, pl.ds(k * L, L)] = zf
            return 0
        lax.fori_loop(0, NROW, zacc, 0)

        def zden(i, _):
            denom[pl.ds(i * L, L)] = zf
            return 0
        lax.fori_loop(0, NROW // L, zden, 0)

        mycnt = cntv[pl.ds(pl.multiple_of(wid * L, 16), L)][0]
        nch = (mycnt + CCH - 1) // CCH

        def alpha_ex(g):
            s = esv[pl.ds(g * L, L)]
            dl = edv[pl.ds(g * L, L)]
            a = (plsc.load_gather(asrc_v, [s]) +
                 plsc.load_gather(adst_v, [dl]))
            a = jnp.where(a > 0, a, a * jnp.float32(0.2))
            return jnp.exp(a), dl

        def chunk1(ch, _):
            base = ch * CCH
            pltpu.sync_copy(es_h.at[pl.ds(pl.multiple_of(wid * CAP + base, 16), CCH)], esv.at[pl.ds(0, CCH)])
            pltpu.sync_copy(ed_h.at[pl.ds(pl.multiple_of(wid * CAP + base, 16), CCH)], edv.at[pl.ds(0, CCH)])
            ngr = jnp.minimum(CCH, mycnt - base) // L

            def group(g, _):
                ex, dl = alpha_ex(g)
                plsc.addupdate_scatter(denom, [dl], ex)
                return 0
            lax.fori_loop(0, ngr, group, 0)
            return 0
        lax.fori_loop(0, nch, chunk1, 0)

        def chunk2(ch, _):
            base = ch * CCH
            pltpu.sync_copy(es_h.at[pl.ds(pl.multiple_of(wid * CAP + base, 16), CCH)], esv.at[pl.ds(0, CCH)])
            pltpu.sync_copy(ed_h.at[pl.ds(pl.multiple_of(wid * CAP + base, 16), CCH)], edv.at[pl.ds(0, CCH)])
            if has_w:
                pltpu.sync_copy(ew_h.at[pl.ds(pl.multiple_of(wid * CAP + base, 16), CCH)], ewv.at[pl.ds(0, CCH)])
            ne_ch = jnp.minimum(CCH, mycnt - base)
            ngr = ne_ch // L

            def group(g, _):
                ex, dl = alpha_ex(g)
                dn = plsc.load_gather(denom, [dl])
                coef = ex / (dn + jnp.float32(1e-16))
                if has_w:
                    coef = coef * ewv[pl.ds(g * L, L)]
                coefv[pl.ds(g * L, L)] = coef
                return 0
            lax.fori_loop(0, ngr, group, 0)

            nblk = (ne_ch + BLK - 1) // BLK

            def issue(bb, buf, sm):
                pltpu.async_copy(
                    h_h.at[esv.at[pl.ds(bb * BLK, BLK)]], rows.at[buf], sm)

            def drain(buf, sm):
                pltpu.make_async_copy(
                    h_h.at[esv.at[pl.ds(0, BLK)]], rows.at[buf], sm).wait()

            sems = (sem, sem2, sem3, sem4)
            for i in range(3):
                @pl.when(i < nblk)
                def _(i=i):
                    issue(i, i, sems[i])

            def block(b, _):
                par = b % 4
                for p in range(4):
                    @pl.when(((b + 3) < nblk) & (par == p))
                    def _(p=p):
                        issue(b + 3, (p + 3) % 4, sems[(p + 3) % 4])
                for p in range(4):
                    @pl.when(par == p)
                    def _(p=p):
                        drain(p, sems[p])

                ne = jnp.minimum(BLK, ne_ch - b * BLK)

                def bgroup(gg, _):
                    eb = gg * L
                    cf16 = coefv[pl.ds(b * BLK + eb, L)]
                    dl16 = edv[pl.ds(b * BLK + eb, L)]
                    for e in range(L):
                        cfv = jnp.full((L,), cf16[e])
                        dl = dl16[e]
                        for q in range(C // (2 * L)):
                            s2, kq = divmod(q, 4)
                            ab = rows[par, eb + e, s2, pl.ds(kq * 32, 32)]
                            va, vb = plsc.unpack(
                                ab, format=plsc.PackFormat.INTERLEAVED,
                                preferred_element_type=jnp.float32)
                            plsc.addupdate(
                                acc.at[dl, pl.ds(q * 2 * L, L)], va * cfv)
                            plsc.addupdate(
                                acc.at[dl, pl.ds(q * 2 * L + L, L)], vb * cfv)
                    return 0
                lax.fori_loop(0, ne // L, bgroup, 0)
                return 0
            lax.fori_loop(0, nblk, block, 0)
            return 0
        lax.fori_loop(0, nch, chunk2, 0)

        for jj in range(10):
            pltpu.sync_copy(acc.at[pl.ds(jj * 32, 32), :],
                            out_h.at[pl.ds((jj * 32 + wid) * 32, 32), :])

    out_type = jax.ShapeDtypeStruct((NPAD, C), jnp.float32)
    scratch = [
        pltpu.VMEM((N,), jnp.float32),
        pltpu.VMEM((NROW,), jnp.float32),
        pltpu.VMEM((NROW,), jnp.float32),
        pltpu.VMEM((NROW, C), jnp.float32),
        pltpu.VMEM((CCH + L,), jnp.int32),
        pltpu.VMEM((CCH + L,), jnp.int32),
    ]
    if has_w:
        scratch.append(pltpu.VMEM((CCH + L,), jnp.float32))
    scratch += [
        pltpu.VMEM((CCH + L,), jnp.float32),
        pltpu.VMEM((4, BLK, 2, 128), jnp.bfloat16),
        pltpu.VMEM((NW * L,), jnp.int32),
        pltpu.SemaphoreType.DMA,
        pltpu.SemaphoreType.DMA,
        pltpu.SemaphoreType.DMA,
        pltpu.SemaphoreType.DMA,
    ]
    return pl.kernel(body, out_type=out_type, mesh=_mesh(),
                     compiler_params=pltpu.CompilerParams(
                         needs_layout_passes=False,
                         use_tc_tiling_on_sc=False),
                     scratch_types=tuple(scratch))


_make_route = functools.lru_cache(maxsize=None)(_make_route)
_make_conv = functools.lru_cache(maxsize=None)(_make_conv)


_BM = 1000


def _mm_body(x_ref, w_ref, as_ref, ad_ref, b_ref, hbf_ref, als_ref, ald_ref):
    xb = x_ref[...] + b_ref[...]
    h = jnp.dot(xb, w_ref[...], preferred_element_type=jnp.float32)
    hbf_ref[...] = h.astype(jnp.bfloat16).reshape(_BM, 2, 128)
    als_ref[...] = jnp.dot(h, as_ref[...], preferred_element_type=jnp.float32)
    ald_ref[...] = jnp.dot(h, ad_ref[...], preferred_element_type=jnp.float32)


def _mm_alpha(x, W, a_src, a_dst, bias):
    h, als, ald = pl.pallas_call(
        _mm_body,
        grid=(N // _BM,),
        in_specs=[
            pl.BlockSpec((_BM, C), lambda i: (i, 0)),
            pl.BlockSpec((C, C), lambda i: (0, 0)),
            pl.BlockSpec((C, 1), lambda i: (0, 0)),
            pl.BlockSpec((C, 1), lambda i: (0, 0)),
            pl.BlockSpec((1, C), lambda i: (0, 0)),
        ],
        out_specs=[
            pl.BlockSpec((_BM, 2, 128), lambda i: (i, 0, 0)),
            pl.BlockSpec((_BM, 1), lambda i: (i, 0)),
            pl.BlockSpec((_BM, 1), lambda i: (i, 0)),
        ],
        out_shape=[
            jax.ShapeDtypeStruct((N, 2, 128), jnp.bfloat16),
            jax.ShapeDtypeStruct((N, 1), jnp.float32),
            jax.ShapeDtypeStruct((N, 1), jnp.float32),
        ],
    )(x, W, a_src[:, None], a_dst[:, None], bias[None, :])
    return h, als[:, 0], ald[:, 0]


def _pool_head_body(x1_ref, x2_ref, batch_ref, lw0_ref, lb0_ref, lw1_ref,
                    lb1_ref, wout_ref, bout_ref, o_ref, g_acc):
    i = pl.program_id(0)

    @pl.when(i == 0)
    def _():
        g_acc[...] = jnp.zeros_like(g_acc)

    node = x2_ref[...] - x1_ref[...]
    gids = lax.broadcasted_iota(jnp.int32, (G, _BM), 0)
    bb = batch_ref[...].reshape(1, _BM)
    oh = (gids == jnp.broadcast_to(bb, (G, _BM))).astype(jnp.float32)
    g_acc[...] += jnp.dot(oh, node, preferred_element_type=jnp.float32)

    @pl.when(i == N // _BM - 1)
    def _():
        g = g_acc[...]
        g = jnp.maximum(jnp.dot(g, lw0_ref[...],
                                preferred_element_type=jnp.float32)
                        + lb0_ref[...], 0.0)
        g = jnp.maximum(jnp.dot(g, lw1_ref[...],
                                preferred_element_type=jnp.float32)
                        + lb1_ref[...], 0.0)
        out = jnp.dot(g, wout_ref[...],
                      preferred_element_type=jnp.float32) + bout_ref[...]
        o_ref[...] = jnp.nan_to_num(out)


def _pool_head(x1, x2, batch, LW0, Lb0, LW1, Lb1, WOut, bOut):
    return pl.pallas_call(
        _pool_head_body,
        grid=(N // _BM,),
        in_specs=[
            pl.BlockSpec((_BM, C), lambda i: (i, 0)),
            pl.BlockSpec((_BM, C), lambda i: (i, 0)),
            pl.BlockSpec((1, 1, _BM), lambda i: (i, 0, 0)),
            pl.BlockSpec((C, C), lambda i: (0, 0)),
            pl.BlockSpec((1, C), lambda i: (0, 0)),
            pl.BlockSpec((C, 128), lambda i: (0, 0)),
            pl.BlockSpec((1, 128), lambda i: (0, 0)),
            pl.BlockSpec((128, 1), lambda i: (0, 0)),
            pl.BlockSpec((1, 1), lambda i: (0, 0)),
        ],
        out_specs=pl.BlockSpec((G, 1), lambda i: (0, 0)),
        out_shape=jax.ShapeDtypeStruct((G, 1), jnp.float32),
        scratch_shapes=[pltpu.VMEM((G, C), jnp.float32)],
    )(x1, x2, batch.reshape(N // _BM, 1, _BM), LW0, Lb0[None, :], LW1, Lb1[None, :],
      WOut, bOut[None, :])


def _perm_idx():
    p = np.arange(C)
    f = np.where(p % 32 % 2 == 0, (p % 32) // 2, 16 + (p % 32) // 2)
    return (p // 32) * 32 + f


_PERM = _perm_idx()


def kernel(x, edge_index_1, edge_index_2, edge_weight, batch,
           W0, a_src0, a_dst0, b0, W1, a_src1, a_dst1, b1,
           LW0, Lb0, LW1, Lb1, WOut, bOut):
    src1, dst1 = edge_index_1[0], edge_index_1[1]
    src2, dst2 = edge_index_2[0], edge_index_2[1]

    cnt1, es1, ed1 = _make_route(False)(src1, dst1)
    cnt2, es2, ed2, ew2 = _make_route(True)(src2, dst2, edge_weight)

    zb = jnp.zeros((C,), jnp.float32)
    h0, as0, ad0 = _mm_alpha(x, W0[:, _PERM], a_src0[_PERM], a_dst0[_PERM], zb)
    ad0p = jnp.pad(ad0, (0, NPAD - N))

    o1 = _make_conv(False)(h0, as0, ad0p, cnt1, es1, ed1)
    o2 = _make_conv(True)(h0, as0, ad0p, cnt2, es2, ed2, ew2)

    h11, as11, ad11 = _mm_alpha(o1[:N], W1[:, _PERM], a_src1[_PERM], a_dst1[_PERM], b0)
    h12, as12, ad12 = _mm_alpha(o2[:N], W1[:, _PERM], a_src1[_PERM], a_dst1[_PERM], b0)

    o3 = _make_conv(False)(h11, as11, jnp.pad(ad11, (0, NPAD - N)), cnt1, es1, ed1)
    o4 = _make_conv(True)(h12, as12, jnp.pad(ad12, (0, NPAD - N)), cnt2, es2, ed2, ew2)

    return _pool_head(o3[:N], o4[:N], batch, LW0, Lb0, LW1, Lb1, WOut, bOut)


# speedup vs baseline: 1.0452x; 1.0452x over previous
import functools

import jax
import jax.numpy as jnp
import numpy as np
from jax import lax
from jax.experimental import pallas as pl
from jax.experimental.pallas import tpu as pltpu
from jax.experimental.pallas import tpu_sc as plsc

N = 10000
E = 160000
G = 64
C = 256

NW = 32
L = 16

KR = 4000
QN = 4
EQ = E // QN
NCH_R = EQ // KR
CBUF = 4096
SCAP = 44544
PADROW = 326
NROW = 328
NPAD = 10240
CCH = 1536
BLK = 64

@functools.lru_cache(maxsize=None)
def _mesh():
    return plsc.VectorSubcoreMesh(core_axis_name="c", subcore_axis_name="s")


def _wid():
    return lax.axis_index("s") * 2 + lax.axis_index("c")


def _make_route(has_w):
    def body(*refs):
        if has_w:
            (src_h, dst_h, w_h, cnt_h, es_h, ed_h, ew_h) = refs[:7]
            rest = refs[7:]
            ssrc, sdst, sw = rest[0], rest[1], rest[2]
            csrcs, cdsts, cws = rest[3:7], rest[7:11], rest[11:15]
            cntw = rest[15]
        else:
            (src_h, dst_h, cnt_h, es_h, ed_h) = refs[:5]
            rest = refs[5:]
            ssrc, sdst = rest[0], rest[1]
            csrcs, cdsts = rest[2:6], rest[6:10]
            sw = None
            cws = (None,) * 4
            cntw = rest[10]
        wid = _wid()
        q = wid & 3
        og = wid >> 2
        zi = jnp.zeros((L,), jnp.int32)
        zf = jnp.zeros((L,), jnp.float32)

        def zb(i, _):
            for j in range(4):
                csrcs[j][pl.ds(i * L, L)] = zi
                cdsts[j][pl.ds(i * L, L)] = zi
            return 0
        lax.fori_loop(0, CBUF // L, zb, 0)

        padrow = jnp.full((L,), PADROW, jnp.int32)
        lanes = lax.broadcasted_iota(jnp.int32, (L,), 0)

        def chunk(ch, wr):
            base = pl.multiple_of(q * EQ + ch * KR, 16)
            pltpu.sync_copy(src_h.at[pl.ds(base, KR)], ssrc)
            pltpu.sync_copy(dst_h.at[pl.ds(base, KR)], sdst)
            if has_w:
                pltpu.sync_copy(w_h.at[pl.ds(base, KR)], sw)

            def group(g, poss):
                s = ssrc[pl.ds(g * L, L)]
                d = sdst[pl.ds(g * L, L)]
                rel = ((d >> 5) & 31) - og * 4
                dl = ((d >> 10) << 5) | (d & 31)
                if has_w:
                    wv = sw[pl.ds(g * L, L)]
                out = []
                for j in range(4):
                    m = rel == j
                    rank = plsc.cumsum(m.astype(jnp.int32)) - 1
                    tgt = jnp.where(m, poss[j] + rank, CBUF + lanes)
                    plsc.store_scatter(csrcs[j], [tgt], s)
                    plsc.store_scatter(cdsts[j], [tgt], dl)
                    if has_w:
                        plsc.store_scatter(cws[j], [tgt], wv)
                    c = plsc.all_reduce_population_count(m)[0]
                    out.append(poss[j] + c)
                return tuple(out)

            poss = lax.fori_loop(0, KR // L, group, (0, 0, 0, 0))
            nwr = []
            for j in range(4):
                pos = poss[j]
                pad = (-pos) & 15
                pm = lanes < pad
                tgt = jnp.where(pm, pos + lanes, CBUF + lanes)
                plsc.store_scatter(csrcs[j], [tgt], zi)
                plsc.store_scatter(cdsts[j], [tgt], padrow)
                if has_w:
                    plsc.store_scatter(cws[j], [tgt], zf)
                pos = pos + pad
                slot = (og * 4 + j) * QN + q
                lbase = pl.multiple_of(slot * SCAP + wr[j], 16)
                pltpu.sync_copy(csrcs[j].at[pl.ds(0, CBUF)],
                                es_h.at[pl.ds(lbase, CBUF)])
                pltpu.sync_copy(cdsts[j].at[pl.ds(0, CBUF)],
                                ed_h.at[pl.ds(lbase, CBUF)])
                if has_w:
                    pltpu.sync_copy(cws[j].at[pl.ds(0, CBUF)],
                                    ew_h.at[pl.ds(lbase, CBUF)])
                nwr.append(wr[j] + pos)
            return tuple(nwr)

        totals = lax.fori_loop(0, NCH_R, chunk, (0, 0, 0, 0))
        for j in range(4):
            cntw[...] = jnp.full((L,), totals[j], jnp.int32)
            slot = (og * 4 + j) * QN + q
            pltpu.sync_copy(
                cntw, cnt_h.at[pl.ds(pl.multiple_of(slot * L, 16), L)])

    out_type = [
        jax.ShapeDtypeStruct((NW * QN * L,), jnp.int32),
        jax.ShapeDtypeStruct((NW * QN * SCAP,), jnp.int32),
        jax.ShapeDtypeStruct((NW * QN * SCAP,), jnp.int32),
    ]
    if has_w:
        out_type.append(jax.ShapeDtypeStruct((NW * QN * SCAP,), jnp.float32))
    scratch = [
        pltpu.VMEM((KR,), jnp.int32),
        pltpu.VMEM((KR,), jnp.int32),
    ]
    if has_w:
        scratch.append(pltpu.VMEM((KR,), jnp.float32))
    scratch += [pltpu.VMEM((CBUF + L,), jnp.int32) for _ in range(8)]
    if has_w:
        scratch += [pltpu.VMEM((CBUF + L,), jnp.float32) for _ in range(4)]
    scratch.append(pltpu.VMEM((L,), jnp.int32))
    return pl.kernel(body, out_type=tuple(out_type), mesh=_mesh(),
                     compiler_params=pltpu.CompilerParams(needs_layout_passes=False),
                     scratch_types=tuple(scratch))


def _make_conv(has_w):
    def body(*refs):
        if has_w:
            (h_h, asrc_h, adst_h, cnt_h, es_h, ed_h, ew_h, out_h,
             asrc_v, adst_v, denom, acc, esv, edv, ewv, coefv, rows,
             cntv, sem, sem2) = refs
        else:
            (h_h, asrc_h, adst_h, cnt_h, es_h, ed_h, out_h,
             asrc_v, adst_v, denom, acc, esv, edv, coefv, rows,
             cntv, sem, sem2) = refs
            ewv = ew_h = None
        wid = _wid()
        pltpu.sync_copy(cnt_h, cntv)
        pltpu.sync_copy(asrc_h, asrc_v)
        for jj in range(10):
            pltpu.sync_copy(adst_h.at[pl.ds((jj * 32 + wid) * 32, 32)],
                            adst_v.at[pl.ds(jj * 32, 32)])
        zf = jnp.zeros((L,), jnp.float32)

        def zacc(r, _):
            for k in range(C // L):
                acc[r, pl.ds(k * L, L)] = zf
            return 0
        lax.fori_loop(0, NROW, zacc, 0)

        def zden(i, _):
            denom[pl.ds(i * L, L)] = zf
            return 0
        lax.fori_loop(0, NROW // L, zden, 0)


        def alpha_ex(g):
            s = esv[pl.ds(g * L, L)]
            dl = edv[pl.ds(g * L, L)]
            a = (plsc.load_gather(asrc_v, [s]) +
                 plsc.load_gather(adst_v, [dl]))
            a = jnp.where(a > 0, a, a * jnp.float32(0.2))
            return jnp.exp(a), dl

        def sub1(sub, _):
            slot = wid * QN + sub
            scnt = cntv[pl.ds(pl.multiple_of(slot * L, 16), L)][0]
            sbase = slot * SCAP
            nch = (scnt + CCH - 1) // CCH

            def chunk1(ch, _):
                base = ch * CCH
                off = pl.multiple_of(sbase + base, 16)
                pltpu.sync_copy(es_h.at[pl.ds(off, CCH)], esv.at[pl.ds(0, CCH)])
                pltpu.sync_copy(ed_h.at[pl.ds(off, CCH)], edv.at[pl.ds(0, CCH)])
                ngr = jnp.minimum(CCH, scnt - base) // L

                def group(g, _):
                    ex, dl = alpha_ex(g)
                    plsc.addupdate_scatter(denom, [dl], ex)
                    return 0
                lax.fori_loop(0, ngr, group, 0)
                return 0
            lax.fori_loop(0, nch, chunk1, 0)
            return 0
        lax.fori_loop(0, QN, sub1, 0)

        def sub2(sub, _):
            slot = wid * QN + sub
            scnt = cntv[pl.ds(pl.multiple_of(slot * L, 16), L)][0]
            sbase = slot * SCAP
            nch2 = (scnt + CCH - 1) // CCH

            def chunk2(ch, _):
                base = ch * CCH
                off = pl.multiple_of(sbase + base, 16)
                pltpu.sync_copy(es_h.at[pl.ds(off, CCH)], esv.at[pl.ds(0, CCH)])
                pltpu.sync_copy(ed_h.at[pl.ds(off, CCH)], edv.at[pl.ds(0, CCH)])
                if has_w:
                    pltpu.sync_copy(ew_h.at[pl.ds(off, CCH)], ewv.at[pl.ds(0, CCH)])
                ne_ch = jnp.minimum(CCH, scnt - base)
                ngr = ne_ch // L

                def group(g, _):
                    ex, dl = alpha_ex(g)
                    dn = plsc.load_gather(denom, [dl])
                    coef = ex / (dn + jnp.float32(1e-16))
                    if has_w:
                        coef = coef * ewv[pl.ds(g * L, L)]
                    coefv[pl.ds(g * L, L)] = coef
                    return 0
                lax.fori_loop(0, ngr, group, 0)

                nblk = (ne_ch + BLK - 1) // BLK

                def issue(bb, buf, sm):
                    pltpu.async_copy(
                        h_h.at[esv.at[pl.ds(bb * BLK, BLK)]], rows.at[buf], sm)

                def drain(buf, sm):
                    pltpu.make_async_copy(
                        h_h.at[esv.at[pl.ds(0, BLK)]], rows.at[buf], sm).wait()

                @pl.when(nblk > 0)
                def _():
                    issue(0, 0, sem)

                def block(b, _):
                    par = b & 1

                    @pl.when(((b + 1) < nblk) & (par == 0))
                    def _():
                        issue(b + 1, 1, sem2)

                    @pl.when(((b + 1) < nblk) & (par == 1))
                    def _():
                        issue(b + 1, 0, sem)

                    @pl.when(par == 0)
                    def _():
                        drain(0, sem)

                    @pl.when(par == 1)
                    def _():
                        drain(1, sem2)

                    ne = jnp.minimum(BLK, ne_ch - b * BLK)

                    def bgroup(gg, _):
                        eb = gg * L
                        cf16 = coefv[pl.ds(b * BLK + eb, L)]
                        dl16 = edv[pl.ds(b * BLK + eb, L)]
                        for e in range(L):
                            cfv = jnp.full((L,), cf16[e])
                            dl = dl16[e]
                            for q in range(C // (2 * L)):
                                s2, kq = divmod(q, 4)
                                ab = rows[par, eb + e, s2, pl.ds(kq * 32, 32)]
                                va, vb = plsc.unpack(
                                    ab, format=plsc.PackFormat.INTERLEAVED,
                                    preferred_element_type=jnp.float32)
                                plsc.addupdate(
                                    acc.at[dl, pl.ds(q * 2 * L, L)], va * cfv)
                                plsc.addupdate(
                                    acc.at[dl, pl.ds(q * 2 * L + L, L)], vb * cfv)
                        return 0
                    lax.fori_loop(0, ne // L, bgroup, 0)
                    return 0
                lax.fori_loop(0, nblk, block, 0)
                return 0
                return 0
            lax.fori_loop(0, nch2, chunk2, 0)
            return 0
        lax.fori_loop(0, QN, sub2, 0)

        for jj in range(10):
            pltpu.sync_copy(acc.at[pl.ds(jj * 32, 32), :],
                            out_h.at[pl.ds((jj * 32 + wid) * 32, 32), :])

    out_type = jax.ShapeDtypeStruct((NPAD, C), jnp.float32)
    scratch = [
        pltpu.VMEM((N,), jnp.float32),
        pltpu.VMEM((NROW,), jnp.float32),
        pltpu.VMEM((NROW,), jnp.float32),
        pltpu.VMEM((NROW, C), jnp.float32),
        pltpu.VMEM((CCH + L,), jnp.int32),
        pltpu.VMEM((CCH + L,), jnp.int32),
    ]
    if has_w:
        scratch.append(pltpu.VMEM((CCH + L,), jnp.float32))
    scratch += [
        pltpu.VMEM((CCH + L,), jnp.float32),
        pltpu.VMEM((2, BLK, 2, 128), jnp.bfloat16),
        pltpu.VMEM((NW * QN * L,), jnp.int32),
        pltpu.SemaphoreType.DMA,
        pltpu.SemaphoreType.DMA,
    ]
    return pl.kernel(body, out_type=out_type, mesh=_mesh(),
                     compiler_params=pltpu.CompilerParams(
                         needs_layout_passes=False,
                         use_tc_tiling_on_sc=False),
                     scratch_types=tuple(scratch))


_make_route = functools.lru_cache(maxsize=None)(_make_route)
_make_conv = functools.lru_cache(maxsize=None)(_make_conv)


_BM = 1000


def _mm_body(x_ref, w_ref, as_ref, ad_ref, b_ref, hbf_ref, als_ref, ald_ref):
    xb = x_ref[...] + b_ref[...]
    h = jnp.dot(xb, w_ref[...], preferred_element_type=jnp.float32)
    hbf_ref[...] = h.astype(jnp.bfloat16).reshape(_BM, 2, 128)
    als_ref[...] = jnp.dot(h, as_ref[...], preferred_element_type=jnp.float32)
    ald_ref[...] = jnp.dot(h, ad_ref[...], preferred_element_type=jnp.float32)


def _mm_alpha(x, W, a_src, a_dst, bias):
    h, als, ald = pl.pallas_call(
        _mm_body,
        grid=(N // _BM,),
        in_specs=[
            pl.BlockSpec((_BM, C), lambda i: (i, 0)),
            pl.BlockSpec((C, C), lambda i: (0, 0)),
            pl.BlockSpec((C, 1), lambda i: (0, 0)),
            pl.BlockSpec((C, 1), lambda i: (0, 0)),
            pl.BlockSpec((1, C), lambda i: (0, 0)),
        ],
        out_specs=[
            pl.BlockSpec((_BM, 2, 128), lambda i: (i, 0, 0)),
            pl.BlockSpec((_BM, 1), lambda i: (i, 0)),
            pl.BlockSpec((_BM, 1), lambda i: (i, 0)),
        ],
        out_shape=[
            jax.ShapeDtypeStruct((N, 2, 128), jnp.bfloat16),
            jax.ShapeDtypeStruct((N, 1), jnp.float32),
            jax.ShapeDtypeStruct((N, 1), jnp.float32),
        ],
    )(x, W, a_src[:, None], a_dst[:, None], bias[None, :])
    return h, als[:, 0], ald[:, 0]


def _pool_head_body(x1_ref, x2_ref, batch_ref, lw0_ref, lb0_ref, lw1_ref,
                    lb1_ref, wout_ref, bout_ref, o_ref, g_acc):
    i = pl.program_id(0)

    @pl.when(i == 0)
    def _():
        g_acc[...] = jnp.zeros_like(g_acc)

    node = x2_ref[...] - x1_ref[...]
    gids = lax.broadcasted_iota(jnp.int32, (G, _BM), 0)
    bb = batch_ref[...].reshape(1, _BM)
    oh = (gids == jnp.broadcast_to(bb, (G, _BM))).astype(jnp.float32)
    g_acc[...] += jnp.dot(oh, node, preferred_element_type=jnp.float32)

    @pl.when(i == N // _BM - 1)
    def _():
        g = g_acc[...]
        g = jnp.maximum(jnp.dot(g, lw0_ref[...],
                                preferred_element_type=jnp.float32)
                        + lb0_ref[...], 0.0)
        g = jnp.maximum(jnp.dot(g, lw1_ref[...],
                                preferred_element_type=jnp.float32)
                        + lb1_ref[...], 0.0)
        out = jnp.dot(g, wout_ref[...],
                      preferred_element_type=jnp.float32) + bout_ref[...]
        o_ref[...] = jnp.nan_to_num(out)


def _pool_head(x1, x2, batch, LW0, Lb0, LW1, Lb1, WOut, bOut):
    return pl.pallas_call(
        _pool_head_body,
        grid=(N // _BM,),
        in_specs=[
            pl.BlockSpec((_BM, C), lambda i: (i, 0)),
            pl.BlockSpec((_BM, C), lambda i: (i, 0)),
            pl.BlockSpec((1, 1, _BM), lambda i: (i, 0, 0)),
            pl.BlockSpec((C, C), lambda i: (0, 0)),
            pl.BlockSpec((1, C), lambda i: (0, 0)),
            pl.BlockSpec((C, 128), lambda i: (0, 0)),
            pl.BlockSpec((1, 128), lambda i: (0, 0)),
            pl.BlockSpec((128, 1), lambda i: (0, 0)),
            pl.BlockSpec((1, 1), lambda i: (0, 0)),
        ],
        out_specs=pl.BlockSpec((G, 1), lambda i: (0, 0)),
        out_shape=jax.ShapeDtypeStruct((G, 1), jnp.float32),
        scratch_shapes=[pltpu.VMEM((G, C), jnp.float32)],
    )(x1, x2, batch.reshape(N // _BM, 1, _BM), LW0, Lb0[None, :], LW1, Lb1[None, :],
      WOut, bOut[None, :])


def _perm_idx():
    p = np.arange(C)
    f = np.where(p % 32 % 2 == 0, (p % 32) // 2, 16 + (p % 32) // 2)
    return (p // 32) * 32 + f


_PERM = _perm_idx()


def kernel(x, edge_index_1, edge_index_2, edge_weight, batch,
           W0, a_src0, a_dst0, b0, W1, a_src1, a_dst1, b1,
           LW0, Lb0, LW1, Lb1, WOut, bOut):
    src1, dst1 = edge_index_1[0], edge_index_1[1]
    src2, dst2 = edge_index_2[0], edge_index_2[1]

    cnt1, es1, ed1 = _make_route(False)(src1, dst1)
    cnt2, es2, ed2, ew2 = _make_route(True)(src2, dst2, edge_weight)

    zb = jnp.zeros((C,), jnp.float32)
    h0, as0, ad0 = _mm_alpha(x, W0[:, _PERM], a_src0[_PERM], a_dst0[_PERM], zb)
    ad0p = jnp.pad(ad0, (0, NPAD - N))

    o1 = _make_conv(False)(h0, as0, ad0p, cnt1, es1, ed1)
    o2 = _make_conv(True)(h0, as0, ad0p, cnt2, es2, ed2, ew2)

    h11, as11, ad11 = _mm_alpha(o1[:N], W1[:, _PERM], a_src1[_PERM], a_dst1[_PERM], b0)
    h12, as12, ad12 = _mm_alpha(o2[:N], W1[:, _PERM], a_src1[_PERM], a_dst1[_PERM], b0)

    o3 = _make_conv(False)(h11, as11, jnp.pad(ad11, (0, NPAD - N)), cnt1, es1, ed1)
    o4 = _make_conv(True)(h12, as12, jnp.pad(ad12, (0, NPAD - N)), cnt2, es2, ed2, ew2)

    return _pool_head(o3[:N], o4[:N], batch, LW0, Lb0, LW1, Lb1, WOut, bOut)
